# baseline (speedup 1.0000x reference)
import functools

import jax
import jax.numpy as jnp
from jax import lax
from jax.experimental import pallas as pl
from jax.experimental.pallas import tpu as pltpu
from jax.experimental.pallas import tpu_sc as plsc

N = 10000
E = 320000
D = 128

NC = 2
NS = 16
NW = NC * NS
CH = 128
CPW = 80
NCHUNKS = NW * CPW
EP = NCHUNKS * CH
ACC_N = 10112
RPT = ACC_N // NS
DGW = 128

_MESH = plsc.VectorSubcoreMesh(core_axis_name="c", subcore_axis_name="s")


@functools.partial(
    pl.kernel,
    out_type=jax.ShapeDtypeStruct((NC, ACC_N, D), jnp.float32),
    mesh=_MESH,
    scratch_types=[
        pltpu.VMEM((CPW, CH), jnp.int32),
        pltpu.VMEM((1, CH), jnp.int32),
        pltpu.VMEM((1, CH), jnp.int32),
        pltpu.VMEM((CH, D), jnp.float32),
        pltpu.VMEM((CH, D), jnp.float32),
        pltpu.VMEM_SHARED((ACC_N, D), jnp.float32),
        pltpu.SemaphoreType.DMA,
        pltpu.SemaphoreType.DMA,
        pltpu.SemaphoreType.DMA,
        pltpu.SemaphoreType.DMA,
        pltpu.SemaphoreType.DMA,
        pltpu.SemaphoreType.DMA,
    ],
)
def _agg(x_hbm, src_hbm, dst_hbm, z2_hbm, acc_out,
         src_v, dst_a, dst_b, rows_a, rows_b, acc_sh,
         sem_a, sem_b, sem_da, sem_db, sem_sa, sem_sb):
    c = lax.axis_index("c")
    s = lax.axis_index("s")
    wid = s * NC + c

    @pl.when(s == 0)
    def _():
        pltpu.sync_copy(z2_hbm, acc_sh)

    base = wid * CPW
    pltpu.sync_copy(src_hbm.at[pl.ds(base, CPW)], src_v)
    plsc.subcore_barrier()

    pltpu.async_copy(dst_hbm.at[base], dst_a, sem_da)
    pltpu.async_copy(dst_hbm.at[base + 1], dst_b, sem_db)
    pltpu.async_copy(x_hbm.at[src_v.at[0]], rows_a, sem_a)
    pltpu.async_copy(x_hbm.at[src_v.at[1]], rows_b, sem_b)
    last = CPW // 2 - 1

    def body(i, carry):
        j0 = 2 * i
        j1 = j0 + 1
        pltpu.make_async_copy(x_hbm.at[src_v.at[j0]], rows_a, sem_a).wait()
        pltpu.make_async_copy(dst_hbm.at[base], dst_a, sem_da).wait()
        pltpu.async_copy(rows_a, acc_sh.at[dst_a.at[0]], sem_sa, add=True)

        pltpu.make_async_copy(x_hbm.at[src_v.at[j1]], rows_b, sem_b).wait()
        pltpu.make_async_copy(dst_hbm.at[base + 1], dst_b, sem_db).wait()
        pltpu.async_copy(rows_b, acc_sh.at[dst_b.at[0]], sem_sb, add=True)

        pltpu.make_async_copy(rows_a, acc_sh.at[dst_a.at[0]], sem_sa).wait()

        @pl.when(i < last)
        def _():
            pltpu.async_copy(dst_hbm.at[base + j0 + 2], dst_a, sem_da)
            pltpu.async_copy(x_hbm.at[src_v.at[j0 + 2]], rows_a, sem_a)

        pltpu.make_async_copy(rows_b, acc_sh.at[dst_b.at[0]], sem_sb).wait()

        @pl.when(i < last)
        def _():
            pltpu.async_copy(dst_hbm.at[base + j1 + 2], dst_b, sem_db)
            pltpu.async_copy(x_hbm.at[src_v.at[j1 + 2]], rows_b, sem_b)

        return carry

    lax.fori_loop(0, CPW // 2, body, 0)
    plsc.subcore_barrier()

    r0 = s * RPT
    pltpu.sync_copy(acc_sh.at[pl.ds(r0, RPT)], acc_out.at[c, pl.ds(r0, RPT)])


@functools.partial(
    pl.kernel,
    out_type=jax.ShapeDtypeStruct((NC, ACC_N, DGW), jnp.float32),
    mesh=_MESH,
    scratch_types=[
        pltpu.VMEM((1, CH), jnp.int32),
        pltpu.VMEM((1, CH), jnp.int32),
        pltpu.VMEM((CH, DGW), jnp.float32),
        pltpu.VMEM_SHARED((ACC_N, DGW), jnp.float32),
        pltpu.SemaphoreType.DMA,
        pltpu.SemaphoreType.DMA,
    ],
)
def _deg(dst_hbm, z1_hbm, one_hbm, deg_out,
         dst_a, dst_b, ones_v, deg_sh, sem_da, sem_db):
    c = lax.axis_index("c")
    s = lax.axis_index("s")
    wid = s * NC + c

    @pl.when(s == 0)
    def _():
        pltpu.sync_copy(z1_hbm, deg_sh)

    base = wid * CPW
    pltpu.sync_copy(one_hbm, ones_v)
    plsc.subcore_barrier()

    pltpu.async_copy(dst_hbm.at[base], dst_a, sem_da)
    pltpu.async_copy(dst_hbm.at[base + 1], dst_b, sem_db)
    last = CPW // 2 - 1

    def body(i, carry):
        j0 = 2 * i
        j1 = j0 + 1
        pltpu.make_async_copy(dst_hbm.at[base], dst_a, sem_da).wait()
        pltpu.sync_copy(ones_v, deg_sh.at[dst_a.at[0]], add=True)

        @pl.when(i < last)
        def _():
            pltpu.async_copy(dst_hbm.at[base + j0 + 2], dst_a, sem_da)

        pltpu.make_async_copy(dst_hbm.at[base + 1], dst_b, sem_db).wait()
        pltpu.sync_copy(ones_v, deg_sh.at[dst_b.at[0]], add=True)

        @pl.when(i < last)
        def _():
            pltpu.async_copy(dst_hbm.at[base + j1 + 2], dst_b, sem_db)

        return carry

    lax.fori_loop(0, CPW // 2, body, 0)
    plsc.subcore_barrier()

    r0 = s * RPT
    pltpu.sync_copy(deg_sh.at[pl.ds(r0, RPT)], deg_out.at[c, pl.ds(r0, RPT)])


def _copy_body(x_ref, o_ref):
    o_ref[...] = x_ref[...]


def _copy(xin):
    return pl.pallas_call(
        _copy_body,
        out_shape=jax.ShapeDtypeStruct((N, D), jnp.float32),
    )(xin)


def _root_body(x_ref, wr_ref, o_ref):
    o_ref[...] = jnp.dot(x_ref[...], wr_ref[...],
                         preferred_element_type=jnp.float32)


def _root(xin, wrT):
    return pl.pallas_call(
        _root_body,
        out_shape=jax.ShapeDtypeStruct((N, D), jnp.float32),
    )(xin, wrT)


def _dense_body(p_ref, deg_ref, xr_ref, wl_ref, bl_ref, o_ref):
    agg = p_ref[0, :N, :] + p_ref[1, :N, :]
    deg = deg_ref[0, :N, 0] + deg_ref[1, :N, 0]
    inv = 1.0 / jnp.maximum(deg, 1.0)
    agg = agg * inv[:, None]
    y = jnp.dot(agg, wl_ref[...], preferred_element_type=jnp.float32)
    y = y + bl_ref[...] + xr_ref[...]
    o_ref[...] = jnp.maximum(y, 0.0)


def _dense(p, degp, xr, wlT, bl2):
    return pl.pallas_call(
        _dense_body,
        out_shape=jax.ShapeDtypeStruct((N, D), jnp.float32),
    )(p, degp, xr, wlT, bl2)


def kernel(x, edge_index, Wl0, bl0, Wr0, Wl1, bl1, Wr1):
    src = edge_index[0]
    dst = edge_index[1]
    pad = EP - E
    src_p = jnp.concatenate(
        [src, jnp.zeros((pad,), jnp.int32)]).reshape(NCHUNKS, CH)
    dst_p = jnp.concatenate(
        [dst, jnp.full((pad,), N, jnp.int32)]).reshape(NCHUNKS, 1, CH)
    z2 = jnp.zeros((ACC_N, D), jnp.float32)
    z1 = jnp.zeros((ACC_N, DGW), jnp.float32)
    ones = jnp.ones((CH, DGW), jnp.float32)

    degp = _deg(dst_p, z1, ones)
    xc = _copy(x)
    p0 = _agg(xc, src_p, dst_p, z2)
    xr0 = _root(x, Wr0.T)
    h = _dense(p0, degp, xr0, Wl0.T, bl0.reshape(1, D))
    p1 = _agg(h, src_p, dst_p, z2)
    xr1 = _root(h, Wr1.T)
    out = _dense(p1, degp, xr1, Wl1.T, bl1.reshape(1, D))
    return out


# speedup vs baseline: 1.2490x; 1.2490x over previous
import functools

import jax
import jax.numpy as jnp
from jax import lax
from jax.experimental import pallas as pl
from jax.experimental.pallas import tpu as pltpu
from jax.experimental.pallas import tpu_sc as plsc

N = 10000
E = 320000
D = 128

NC = 2
NS = 16
NW = NC * NS
CH = 128
CPW = 80
NCHUNKS = NW * CPW
EP = NCHUNKS * CH
CPW_F = 120
CPW_S = 40
GROUP = CPW_F + CPW_S
C_FAST = 0
ACC_N = 10112
RPT = ACC_N // NS
DGW = 128

_MESH = plsc.VectorSubcoreMesh(core_axis_name="c", subcore_axis_name="s")


@functools.partial(
    pl.kernel,
    out_type=jax.ShapeDtypeStruct((NC, ACC_N, D), jnp.float32),
    mesh=_MESH,
    scratch_types=[
        pltpu.VMEM((CPW_F, CH), jnp.int32),
        pltpu.VMEM((1, CH), jnp.int32),
        pltpu.VMEM((1, CH), jnp.int32),
        pltpu.VMEM((CH, D), jnp.float32),
        pltpu.VMEM((CH, D), jnp.float32),
        pltpu.VMEM_SHARED((ACC_N, D), jnp.float32),
        pltpu.SemaphoreType.DMA,
        pltpu.SemaphoreType.DMA,
        pltpu.SemaphoreType.DMA,
        pltpu.SemaphoreType.DMA,
    ],
)
def _agg(x_hbm, src_hbm, dst_hbm, z2_hbm, acc_out,
         src_v, dst_a, dst_b, rows_a, rows_b, acc_sh,
         sem_a, sem_b, sem_da, sem_db):
    c = lax.axis_index("c")
    s = lax.axis_index("s")

    @pl.when(s == 0)
    def _():
        pltpu.sync_copy(z2_hbm, acc_sh)

    on_fast = c == C_FAST
    base = s * GROUP + jnp.where(on_fast, 0, CPW_F)
    ncpw = jnp.where(on_fast, CPW_F, CPW_S)

    @pl.when(on_fast)
    def _():
        pltpu.sync_copy(src_hbm.at[pl.ds(s * GROUP, CPW_F)], src_v)

    @pl.when(jnp.logical_not(on_fast))
    def _():
        pltpu.sync_copy(src_hbm.at[pl.ds(s * GROUP + CPW_F, CPW_S)],
                        src_v.at[pl.ds(0, CPW_S)])

    plsc.subcore_barrier()

    pltpu.async_copy(dst_hbm.at[base], dst_a, sem_da)
    pltpu.async_copy(dst_hbm.at[base + 1], dst_b, sem_db)
    pltpu.async_copy(x_hbm.at[src_v.at[0]], rows_a, sem_a)
    last = ncpw // 2 - 1

    def body(i, carry):
        j0 = 2 * i
        j1 = j0 + 1
        pltpu.make_async_copy(x_hbm.at[src_v.at[j0]], rows_a, sem_a).wait()
        pltpu.async_copy(x_hbm.at[src_v.at[j1]], rows_b, sem_b)
        pltpu.make_async_copy(dst_hbm.at[base], dst_a, sem_da).wait()
        pltpu.sync_copy(rows_a, acc_sh.at[dst_a.at[0]], add=True)

        @pl.when(i < last)
        def _():
            pltpu.async_copy(dst_hbm.at[base + j0 + 2], dst_a, sem_da)

        pltpu.make_async_copy(x_hbm.at[src_v.at[j1]], rows_b, sem_b).wait()

        @pl.when(i < last)
        def _():
            pltpu.async_copy(x_hbm.at[src_v.at[j0 + 2]], rows_a, sem_a)

        pltpu.make_async_copy(dst_hbm.at[base + 1], dst_b, sem_db).wait()
        pltpu.sync_copy(rows_b, acc_sh.at[dst_b.at[0]], add=True)

        @pl.when(i < last)
        def _():
            pltpu.async_copy(dst_hbm.at[base + j1 + 2], dst_b, sem_db)

        return carry

    lax.fori_loop(0, ncpw // 2, body, 0)
    plsc.subcore_barrier()

    r0 = s * RPT
    pltpu.sync_copy(acc_sh.at[pl.ds(r0, RPT)], acc_out.at[c, pl.ds(r0, RPT)])


@functools.partial(
    pl.kernel,
    out_type=jax.ShapeDtypeStruct((NC, ACC_N, DGW), jnp.float32),
    mesh=_MESH,
    scratch_types=[
        pltpu.VMEM((1, CH), jnp.int32),
        pltpu.VMEM((1, CH), jnp.int32),
        pltpu.VMEM((CH, DGW), jnp.float32),
        pltpu.VMEM_SHARED((ACC_N, DGW), jnp.float32),
        pltpu.SemaphoreType.DMA,
        pltpu.SemaphoreType.DMA,
    ],
)
def _deg(dst_hbm, z1_hbm, one_hbm, deg_out,
         dst_a, dst_b, ones_v, deg_sh, sem_da, sem_db):
    c = lax.axis_index("c")
    s = lax.axis_index("s")
    wid = s * NC + c

    @pl.when(s == 0)
    def _():
        pltpu.sync_copy(z1_hbm, deg_sh)

    base = wid * CPW
    pltpu.sync_copy(one_hbm, ones_v)
    plsc.subcore_barrier()

    pltpu.async_copy(dst_hbm.at[base], dst_a, sem_da)
    pltpu.async_copy(dst_hbm.at[base + 1], dst_b, sem_db)
    last = CPW // 2 - 1

    def body(i, carry):
        j0 = 2 * i
        j1 = j0 + 1
        pltpu.make_async_copy(dst_hbm.at[base], dst_a, sem_da).wait()
        pltpu.sync_copy(ones_v, deg_sh.at[dst_a.at[0]], add=True)

        @pl.when(i < last)
        def _():
            pltpu.async_copy(dst_hbm.at[base + j0 + 2], dst_a, sem_da)

        pltpu.make_async_copy(dst_hbm.at[base + 1], dst_b, sem_db).wait()
        pltpu.sync_copy(ones_v, deg_sh.at[dst_b.at[0]], add=True)

        @pl.when(i < last)
        def _():
            pltpu.async_copy(dst_hbm.at[base + j1 + 2], dst_b, sem_db)

        return carry

    lax.fori_loop(0, CPW // 2, body, 0)
    plsc.subcore_barrier()

    r0 = s * RPT
    pltpu.sync_copy(deg_sh.at[pl.ds(r0, RPT)], deg_out.at[c, pl.ds(r0, RPT)])


def _copy_body(x_ref, o_ref):
    o_ref[...] = x_ref[...]


def _copy(xin):
    return pl.pallas_call(
        _copy_body,
        out_shape=jax.ShapeDtypeStruct((N, D), jnp.float32),
    )(xin)


def _root_body(x_ref, wr_ref, o_ref):
    o_ref[...] = jnp.dot(x_ref[...], wr_ref[...],
                         preferred_element_type=jnp.float32)


def _root(xin, wrT):
    return pl.pallas_call(
        _root_body,
        out_shape=jax.ShapeDtypeStruct((N, D), jnp.float32),
    )(xin, wrT)


def _dense_body(p_ref, deg_ref, xr_ref, wl_ref, bl_ref, o_ref):
    agg = p_ref[0, :N, :] + p_ref[1, :N, :]
    deg = deg_ref[0, :N, 0] + deg_ref[1, :N, 0]
    inv = 1.0 / jnp.maximum(deg, 1.0)
    agg = agg * inv[:, None]
    y = jnp.dot(agg, wl_ref[...], preferred_element_type=jnp.float32)
    y = y + bl_ref[...] + xr_ref[...]
    o_ref[...] = jnp.maximum(y, 0.0)


def _dense(p, degp, xr, wlT, bl2):
    return pl.pallas_call(
        _dense_body,
        out_shape=jax.ShapeDtypeStruct((N, D), jnp.float32),
    )(p, degp, xr, wlT, bl2)


def kernel(x, edge_index, Wl0, bl0, Wr0, Wl1, bl1, Wr1):
    src = edge_index[0]
    dst = edge_index[1]
    pad = EP - E
    src_p = jnp.concatenate(
        [src, jnp.zeros((pad,), jnp.int32)]).reshape(NCHUNKS, CH)
    dst_p = jnp.concatenate(
        [dst, jnp.full((pad,), N, jnp.int32)]).reshape(NCHUNKS, 1, CH)
    z2 = jnp.zeros((ACC_N, D), jnp.float32)
    z1 = jnp.zeros((ACC_N, DGW), jnp.float32)
    ones = jnp.ones((CH, DGW), jnp.float32)

    degp = _deg(dst_p, z1, ones)
    xc = _copy(x)
    p0 = _agg(xc, src_p, dst_p, z2)
    xr0 = _root(x, Wr0.T)
    h = _dense(p0, degp, xr0, Wl0.T, bl0.reshape(1, D))
    p1 = _agg(h, src_p, dst_p, z2)
    xr1 = _root(h, Wr1.T)
    out = _dense(p1, degp, xr1, Wl1.T, bl1.reshape(1, D))
    return out


# speedup vs baseline: 1.2541x; 1.0041x over previous
import functools

import jax
import jax.numpy as jnp
from jax import lax
from jax.experimental import pallas as pl
from jax.experimental.pallas import tpu as pltpu
from jax.experimental.pallas import tpu_sc as plsc

N = 10000
E = 320000
D = 128

NC = 2
NS = 16
NW = NC * NS
CH = 128
CPW = 80
NCHUNKS = NW * CPW
EP = NCHUNKS * CH
CPW_F = 120
CPW_S = 40
GROUP = CPW_F + CPW_S
C_FAST = 0
ACC_N = 10112
RPT = ACC_N // NS
DGW = 128

_MESH = plsc.VectorSubcoreMesh(core_axis_name="c", subcore_axis_name="s")


@functools.partial(
    pl.kernel,
    out_type=jax.ShapeDtypeStruct((NC, ACC_N, D), jnp.float32),
    mesh=_MESH,
    scratch_types=[
        pltpu.VMEM((CPW_F, CH), jnp.int32),
        pltpu.VMEM((1, CH), jnp.int32),
        pltpu.VMEM((1, CH), jnp.int32),
        pltpu.VMEM((CH, D), jnp.float32),
        pltpu.VMEM((CH, D), jnp.float32),
        pltpu.VMEM_SHARED((ACC_N, D), jnp.float32),
        pltpu.SemaphoreType.DMA,
        pltpu.SemaphoreType.DMA,
        pltpu.SemaphoreType.DMA,
        pltpu.SemaphoreType.DMA,
    ],
)
def _agg(x_hbm, src_hbm, dst_hbm, z2_hbm, acc_out,
         src_v, dst_a, dst_b, rows_a, rows_b, acc_sh,
         sem_a, sem_b, sem_da, sem_db):
    c = lax.axis_index("c")
    s = lax.axis_index("s")

    @pl.when(s == 0)
    def _():
        pltpu.sync_copy(z2_hbm, acc_sh)

    on_fast = c == C_FAST
    base = s * GROUP + jnp.where(on_fast, 0, CPW_F)
    ncpw = jnp.where(on_fast, CPW_F, CPW_S)

    @pl.when(on_fast)
    def _():
        pltpu.sync_copy(src_hbm.at[pl.ds(s * GROUP, CPW_F)], src_v)

    @pl.when(jnp.logical_not(on_fast))
    def _():
        pltpu.sync_copy(src_hbm.at[pl.ds(s * GROUP + CPW_F, CPW_S)],
                        src_v.at[pl.ds(0, CPW_S)])

    plsc.subcore_barrier()

    HF = CH // 2

    def gather(j, buf, sem):
        pltpu.async_copy(x_hbm.at[src_v.at[j, pl.ds(0, HF)]],
                         buf.at[pl.ds(0, HF)], sem)
        pltpu.async_copy(x_hbm.at[src_v.at[j, pl.ds(HF, HF)]],
                         buf.at[pl.ds(HF, HF)], sem)

    def gather_wait(j, buf, sem):
        pltpu.make_async_copy(x_hbm.at[src_v.at[j, pl.ds(0, HF)]],
                              buf.at[pl.ds(0, HF)], sem).wait()
        pltpu.make_async_copy(x_hbm.at[src_v.at[j, pl.ds(HF, HF)]],
                              buf.at[pl.ds(HF, HF)], sem).wait()

    pltpu.async_copy(dst_hbm.at[base], dst_a, sem_da)
    pltpu.async_copy(dst_hbm.at[base + 1], dst_b, sem_db)
    gather(0, rows_a, sem_a)
    last = ncpw // 2 - 1

    def body(i, carry):
        j0 = 2 * i
        j1 = j0 + 1
        gather_wait(j0, rows_a, sem_a)
        gather(j1, rows_b, sem_b)
        pltpu.make_async_copy(dst_hbm.at[base], dst_a, sem_da).wait()
        pltpu.sync_copy(rows_a, acc_sh.at[dst_a.at[0]], add=True)

        @pl.when(i < last)
        def _():
            pltpu.async_copy(dst_hbm.at[base + j0 + 2], dst_a, sem_da)

        gather_wait(j1, rows_b, sem_b)

        @pl.when(i < last)
        def _():
            gather(j0 + 2, rows_a, sem_a)

        pltpu.make_async_copy(dst_hbm.at[base + 1], dst_b, sem_db).wait()
        pltpu.sync_copy(rows_b, acc_sh.at[dst_b.at[0]], add=True)

        @pl.when(i < last)
        def _():
            pltpu.async_copy(dst_hbm.at[base + j1 + 2], dst_b, sem_db)

        return carry

    lax.fori_loop(0, ncpw // 2, body, 0)
    plsc.subcore_barrier()

    r0 = s * RPT
    pltpu.sync_copy(acc_sh.at[pl.ds(r0, RPT)], acc_out.at[c, pl.ds(r0, RPT)])


@functools.partial(
    pl.kernel,
    out_type=jax.ShapeDtypeStruct((NC, ACC_N, DGW), jnp.float32),
    mesh=_MESH,
    scratch_types=[
        pltpu.VMEM((1, CH), jnp.int32),
        pltpu.VMEM((1, CH), jnp.int32),
        pltpu.VMEM((CH, DGW), jnp.float32),
        pltpu.VMEM_SHARED((ACC_N, DGW), jnp.float32),
        pltpu.SemaphoreType.DMA,
        pltpu.SemaphoreType.DMA,
    ],
)
def _deg(dst_hbm, z1_hbm, one_hbm, deg_out,
         dst_a, dst_b, ones_v, deg_sh, sem_da, sem_db):
    c = lax.axis_index("c")
    s = lax.axis_index("s")
    wid = s * NC + c

    @pl.when(s == 0)
    def _():
        pltpu.sync_copy(z1_hbm, deg_sh)

    base = wid * CPW
    pltpu.sync_copy(one_hbm, ones_v)
    plsc.subcore_barrier()

    pltpu.async_copy(dst_hbm.at[base], dst_a, sem_da)
    pltpu.async_copy(dst_hbm.at[base + 1], dst_b, sem_db)
    last = CPW // 2 - 1

    def body(i, carry):
        j0 = 2 * i
        j1 = j0 + 1
        pltpu.make_async_copy(dst_hbm.at[base], dst_a, sem_da).wait()
        pltpu.sync_copy(ones_v, deg_sh.at[dst_a.at[0]], add=True)

        @pl.when(i < last)
        def _():
            pltpu.async_copy(dst_hbm.at[base + j0 + 2], dst_a, sem_da)

        pltpu.make_async_copy(dst_hbm.at[base + 1], dst_b, sem_db).wait()
        pltpu.sync_copy(ones_v, deg_sh.at[dst_b.at[0]], add=True)

        @pl.when(i < last)
        def _():
            pltpu.async_copy(dst_hbm.at[base + j1 + 2], dst_b, sem_db)

        return carry

    lax.fori_loop(0, CPW // 2, body, 0)
    plsc.subcore_barrier()

    r0 = s * RPT
    pltpu.sync_copy(deg_sh.at[pl.ds(r0, RPT)], deg_out.at[c, pl.ds(r0, RPT)])


def _copy_body(x_ref, o_ref):
    o_ref[...] = x_ref[...]


def _copy(xin):
    return pl.pallas_call(
        _copy_body,
        out_shape=jax.ShapeDtypeStruct((N, D), jnp.float32),
    )(xin)


def _root_body(x_ref, wr_ref, o_ref):
    o_ref[...] = jnp.dot(x_ref[...], wr_ref[...],
                         preferred_element_type=jnp.float32)


def _root(xin, wrT):
    return pl.pallas_call(
        _root_body,
        out_shape=jax.ShapeDtypeStruct((N, D), jnp.float32),
    )(xin, wrT)


def _dense_body(p_ref, deg_ref, xr_ref, wl_ref, bl_ref, o_ref):
    agg = p_ref[0, :N, :] + p_ref[1, :N, :]
    deg = deg_ref[0, :N, 0] + deg_ref[1, :N, 0]
    inv = 1.0 / jnp.maximum(deg, 1.0)
    agg = agg * inv[:, None]
    y = jnp.dot(agg, wl_ref[...], preferred_element_type=jnp.float32)
    y = y + bl_ref[...] + xr_ref[...]
    o_ref[...] = jnp.maximum(y, 0.0)


def _dense(p, degp, xr, wlT, bl2):
    return pl.pallas_call(
        _dense_body,
        out_shape=jax.ShapeDtypeStruct((N, D), jnp.float32),
    )(p, degp, xr, wlT, bl2)


def kernel(x, edge_index, Wl0, bl0, Wr0, Wl1, bl1, Wr1):
    src = edge_index[0]
    dst = edge_index[1]
    pad = EP - E
    src_p = jnp.concatenate(
        [src, jnp.zeros((pad,), jnp.int32)]).reshape(NCHUNKS, CH)
    dst_p = jnp.concatenate(
        [dst, jnp.full((pad,), N, jnp.int32)]).reshape(NCHUNKS, 1, CH)
    z2 = jnp.zeros((ACC_N, D), jnp.float32)
    z1 = jnp.zeros((ACC_N, DGW), jnp.float32)
    ones = jnp.ones((CH, DGW), jnp.float32)

    degp = _deg(dst_p, z1, ones)
    xc = _copy(x)
    p0 = _agg(xc, src_p, dst_p, z2)
    xr0 = _root(x, Wr0.T)
    h = _dense(p0, degp, xr0, Wl0.T, bl0.reshape(1, D))
    p1 = _agg(h, src_p, dst_p, z2)
    xr1 = _root(h, Wr1.T)
    out = _dense(p1, degp, xr1, Wl1.T, bl1.reshape(1, D))
    return out
